# baseline (speedup 1.0000x reference)
import functools

import jax
import jax.numpy as jnp
from jax import lax
from jax.experimental import pallas as pl
from jax.experimental.pallas import tpu as pltpu
from jax.experimental.pallas import tpu_sc as plsc

_B = 16384
_D = 64
_CHUNK = 128
_NC = 2
_NS = 16
_NW = _NC * _NS
_BPW = _B // _NW
_NCHUNK = _BPW // _CHUNK


@functools.cache
def _build():
    mesh = plsc.VectorSubcoreMesh(core_axis_name="c", subcore_axis_name="s")

    @functools.partial(
        pl.kernel,
        mesh=mesh,
        out_type=jax.ShapeDtypeStruct((_B, _D), jnp.float32),
        scratch_types=[
            pltpu.VMEM((_NCHUNK, _CHUNK), jnp.int32),
            pltpu.VMEM((_BPW, _D), jnp.float32),
            pltpu.SemaphoreType.DMA,
        ],
        compiler_params=pltpu.CompilerParams(use_tc_tiling_on_sc=False),
    )
    def gather_kernel(table_hbm, idx_hbm, out_hbm, idx_v, rows_v, sem):
        wid = lax.axis_index("s") * _NC + lax.axis_index("c")
        pltpu.sync_copy(idx_hbm.at[pl.ds(wid * _NCHUNK, _NCHUNK)], idx_v)
        copies = []
        for j in range(_NCHUNK):
            copies.append(
                pltpu.async_copy(
                    table_hbm.at[idx_v.at[j]],
                    rows_v.at[pl.ds(j * _CHUNK, _CHUNK)],
                    sem,
                )
            )
        for c in copies:
            c.wait()
        pltpu.sync_copy(rows_v, out_hbm.at[pl.ds(wid * _BPW, _BPW)])

    return gather_kernel


def kernel(memory, index, t, current_event_id):
    idx2d = index.astype(jnp.int32).reshape(_B // _CHUNK, _CHUNK)
    return _build()(memory, idx2d)


# speedup vs baseline: 1.7288x; 1.7288x over previous
import functools

import jax
import jax.numpy as jnp
from jax import lax
from jax.experimental import pallas as pl
from jax.experimental.pallas import tpu as pltpu
from jax.experimental.pallas import tpu_sc as plsc

_B = 16384
_D = 64
_NC = 2
_NS = 16
_NW = _NC * _NS
_BPW = _B // _NW


@functools.cache
def _build():
    mesh = plsc.VectorSubcoreMesh(core_axis_name="c", subcore_axis_name="s")

    @functools.partial(
        pl.kernel,
        mesh=mesh,
        out_type=jax.ShapeDtypeStruct((_B, _D), jnp.float32),
        scratch_types=[
            pltpu.VMEM((_BPW,), jnp.int32),
            pltpu.VMEM((_BPW, _D), jnp.float32),
            pltpu.SemaphoreType.DMA,
        ],
    )
    def gather_kernel(table_hbm, idx_hbm, out_hbm, idx_v, rows_v, sem):
        wid = lax.axis_index("s") * _NC + lax.axis_index("c")
        base = wid * _BPW
        pltpu.sync_copy(idx_hbm.at[pl.ds(base, _BPW)], idx_v)

        def body(g, carry):
            vec = idx_v[pl.ds(g * 16, 16)]
            for l in range(16):
                pltpu.async_copy(
                    table_hbm.at[vec[l]], rows_v.at[g * 16 + l], sem)
            return carry

        lax.fori_loop(0, _BPW // 16, body, 0)
        pltpu.make_async_copy(
            table_hbm.at[pl.ds(0, _BPW)], rows_v, sem).wait()

        pltpu.sync_copy(rows_v, out_hbm.at[pl.ds(base, _BPW)])

    return gather_kernel


def kernel(memory, index, t, current_event_id):
    return _build()(memory, index.astype(jnp.int32))


# speedup vs baseline: 3.1918x; 1.8462x over previous
import functools

import jax
import jax.numpy as jnp
from jax import lax
from jax.experimental import pallas as pl
from jax.experimental.pallas import tpu as pltpu
from jax.experimental.pallas import tpu_sc as plsc

_B = 16384
_D = 64
_LANES = 128
_NBLK = (1000000 + _LANES - 1) // _LANES
_NC = 2
_NS = 16
_NW = _NC * _NS
_BPW = _B // _NW


@functools.cache
def _build():
    mesh = plsc.VectorSubcoreMesh(core_axis_name="c", subcore_axis_name="s")

    @functools.partial(
        pl.kernel,
        mesh=mesh,
        out_type=jax.ShapeDtypeStruct((_B, _D), jnp.float32),
        scratch_types=[
            pltpu.VMEM((_BPW + 16,), jnp.int32),
            pltpu.VMEM((_BPW + 16,), jnp.int32),
            pltpu.VMEM((_D, _LANES), jnp.float32),
            pltpu.VMEM((_D, _LANES), jnp.float32),
            pltpu.VMEM((_BPW, _D), jnp.float32),
            pltpu.SemaphoreType.DMA,
            pltpu.SemaphoreType.DMA,
            pltpu.SemaphoreType.DMA,
        ],
        compiler_params=pltpu.CompilerParams(needs_layout_passes=False),
    )
    def gather_kernel(tableT_hbm, sidx_hbm, opos_hbm, out_hbm,
                      sidx_v, opos_v, blk0, blk1, rows_v,
                      sem0, sem1, semw):
        wid = lax.axis_index("s") * _NC + lax.axis_index("c")
        base = wid * _BPW
        pltpu.sync_copy(sidx_hbm.at[pl.ds(base, _BPW)],
                        sidx_v.at[pl.ds(0, _BPW)])
        pltpu.sync_copy(opos_hbm.at[pl.ds(base, _BPW)],
                        opos_v.at[pl.ds(0, _BPW)])

        blo = sidx_v[pl.ds(0, 16)][0] >> 7
        bhi = sidx_v[pl.ds(_BPW - 16, 16)][15] >> 7

        lane16 = jnp.arange(16, dtype=jnp.int32)

        def start(b, blk, sem):
            off = pl.multiple_of(b * _LANES, _LANES)
            pltpu.async_copy(tableT_hbm.at[:, pl.ds(off, _LANES)], blk, sem)

        def wait(blk, sem):
            pltpu.make_async_copy(
                tableT_hbm.at[:, pl.ds(0, _LANES)], blk, sem).wait()

        def proc(blk, b, q0):
            lim = (b + 1) * _LANES

            def cond(q):
                v = sidx_v[pl.ds(q, 16)][0]
                return jnp.logical_and(q < _BPW, v < lim)

            def body(q):
                v = sidx_v[pl.ds(q, 16)][0]
                o = opos_v[pl.ds(q, 16)][0]
                lvec = jnp.full((16,), 0, jnp.int32) + (v & (_LANES - 1))
                for g in range(4):
                    col = plsc.load_gather(
                        blk, [lane16 + g * 16, lvec])
                    rows_v[q, pl.ds(g * 16, 16)] = col
                pltpu.async_copy(rows_v.at[q], out_hbm.at[o], semw)
                return q + 1

            return lax.while_loop(cond, body, q0)

        start(blo, blk0, sem0)

        @pl.when(bhi > blo)
        def _():
            start(blo + 1, blk1, sem1)

        def ring(k2, q):
            b0 = blo + 2 * k2
            b1 = b0 + 1
            wait(blk0, sem0)
            q = proc(blk0, b0, q)

            @pl.when(b0 + 2 <= bhi)
            def _():
                start(b0 + 2, blk0, sem0)

            @pl.when(b1 <= bhi)
            def _():
                wait(blk1, sem1)

            q = proc(blk1, b1, q)

            @pl.when(b1 + 2 <= bhi)
            def _():
                start(b1 + 2, blk1, sem1)

            return q

        npairs = (bhi - blo + 2) >> 1
        lax.fori_loop(0, npairs, ring, 0)

        pltpu.make_async_copy(
            out_hbm.at[pl.ds(0, _BPW)], rows_v, semw).wait()

    return gather_kernel


def kernel(memory, index, t, current_event_id):
    idx = index.astype(jnp.int32)
    pos = jnp.arange(_B, dtype=jnp.int32)
    sidx, opos = lax.sort((idx, pos), num_keys=1)
    return _build()(memory.T, sidx, opos)


# speedup vs baseline: 4.8946x; 1.5335x over previous
import functools

import jax
import jax.numpy as jnp
from jax import lax
from jax.experimental import pallas as pl
from jax.experimental.pallas import tpu as pltpu
from jax.experimental.pallas import tpu_sc as plsc

_B = 16384
_D = 64
_LANES = 128
_NC = 2
_NS = 16
_NW = _NC * _NS
_BPW = _B // _NW
_RING = 6


@functools.cache
def _build():
    mesh = plsc.VectorSubcoreMesh(core_axis_name="c", subcore_axis_name="s")

    @functools.partial(
        pl.kernel,
        mesh=mesh,
        out_type=jax.ShapeDtypeStruct((_B, _D), jnp.float32),
        scratch_types=[
            pltpu.VMEM((_BPW + 16,), jnp.int32),
            pltpu.VMEM((_BPW + 16,), jnp.int32),
            pltpu.VMEM((_RING, _D, _LANES), jnp.float32),
            pltpu.VMEM((_BPW, _D), jnp.float32),
            pltpu.SemaphoreType.DMA,
            pltpu.SemaphoreType.DMA,
            pltpu.SemaphoreType.DMA,
            pltpu.SemaphoreType.DMA,
            pltpu.SemaphoreType.DMA,
            pltpu.SemaphoreType.DMA,
            pltpu.SemaphoreType.DMA,
        ],
        compiler_params=pltpu.CompilerParams(needs_layout_passes=False),
    )
    def gather_kernel(tableT_hbm, sidx_hbm, opos_hbm, out_hbm,
                      sidx_v, opos_v, ring_v, rows_v,
                      s0, s1, s2, s3, s4, s5, semw):
        sems = [s0, s1, s2, s3, s4, s5]
        wid = lax.axis_index("s") * _NC + lax.axis_index("c")
        base = wid * _BPW
        pltpu.sync_copy(sidx_hbm.at[pl.ds(base, _BPW)],
                        sidx_v.at[pl.ds(0, _BPW)])
        pltpu.sync_copy(opos_hbm.at[pl.ds(base, _BPW)],
                        opos_v.at[pl.ds(0, _BPW)])

        blo = sidx_v[pl.ds(0, 16)][0] >> 7
        bhi = sidx_v[pl.ds(_BPW - 16, 16)][15] >> 7

        lane16 = jnp.arange(16, dtype=jnp.int32)

        def start(b, i):
            off = pl.multiple_of(b * _LANES, _LANES)
            pltpu.async_copy(tableT_hbm.at[:, pl.ds(off, _LANES)],
                             ring_v.at[i], sems[i])

        def wait(i):
            pltpu.make_async_copy(
                tableT_hbm.at[:, pl.ds(0, _LANES)], ring_v.at[i],
                sems[i]).wait()

        def proc(i, b, q0):
            blk = ring_v.at[i]
            lim = (b + 1) * _LANES

            def cond(q):
                v = sidx_v[pl.ds(q, 16)][0]
                return jnp.logical_and(q < _BPW, v < lim)

            def body(q):
                v = sidx_v[pl.ds(q, 16)][0]
                o = opos_v[pl.ds(q, 16)][0]
                lvec = jnp.full((16,), 0, jnp.int32) + (v & (_LANES - 1))
                for g in range(4):
                    col = plsc.load_gather(
                        blk, [lane16 + g * 16, lvec])
                    rows_v[q, pl.ds(g * 16, 16)] = col
                pltpu.async_copy(rows_v.at[q], out_hbm.at[o], semw)
                return q + 1

            return lax.while_loop(cond, body, q0)

        for i in range(_RING):
            @pl.when(blo + i <= bhi)
            def _(i=i):
                start(blo + i, i)

        def rotate(k, q):
            b0 = blo + k * _RING
            for i in range(_RING):
                b = b0 + i

                @pl.when(b <= bhi)
                def _(i=i):
                    wait(i)

                q = proc(i, b, q)

                @pl.when(b + _RING <= bhi)
                def _(i=i, b=b):
                    start(b + _RING, i)

            return q

        nrot = (bhi - blo + _RING) // _RING
        lax.fori_loop(0, nrot, rotate, 0)

        pltpu.make_async_copy(
            out_hbm.at[pl.ds(0, _BPW)], rows_v, semw).wait()

    return gather_kernel


def kernel(memory, index, t, current_event_id):
    idx = index.astype(jnp.int32)
    pos = jnp.arange(_B, dtype=jnp.int32)
    sidx, opos = lax.sort((idx, pos), num_keys=1)
    return _build()(memory.T, sidx, opos)


# speedup vs baseline: 4.9741x; 1.0163x over previous
import functools

import jax
import jax.numpy as jnp
from jax import lax
from jax.experimental import pallas as pl
from jax.experimental.pallas import tpu as pltpu
from jax.experimental.pallas import tpu_sc as plsc

_B = 16384
_D = 64
_LANES = 128
_NC = 2
_NS = 16
_NW = _NC * _NS
_BPW = _B // _NW
_RING = 7


@functools.cache
def _build():
    mesh = plsc.VectorSubcoreMesh(core_axis_name="c", subcore_axis_name="s")

    @functools.partial(
        pl.kernel,
        mesh=mesh,
        out_type=jax.ShapeDtypeStruct((_B, _D), jnp.float32),
        scratch_types=[
            pltpu.VMEM((_BPW + 16,), jnp.int32),
            pltpu.VMEM((_BPW + 16,), jnp.int32),
            pltpu.VMEM((_RING, _D, _LANES), jnp.float32),
            pltpu.VMEM((_BPW, _D), jnp.float32),
        ] + [pltpu.SemaphoreType.DMA] * (_RING + 1) + [
        ],
        compiler_params=pltpu.CompilerParams(needs_layout_passes=False),
    )
    def gather_kernel(tableT_hbm, sidx_hbm, opos_hbm, out_hbm,
                      sidx_v, opos_v, ring_v, rows_v, *sems_all):
        sems = list(sems_all[:_RING])
        semw = sems_all[_RING]
        wid = lax.axis_index("s") * _NC + lax.axis_index("c")
        base = wid * _BPW
        pltpu.sync_copy(sidx_hbm.at[pl.ds(base, _BPW)],
                        sidx_v.at[pl.ds(0, _BPW)])
        pltpu.sync_copy(opos_hbm.at[pl.ds(base, _BPW)],
                        opos_v.at[pl.ds(0, _BPW)])

        blo = sidx_v[pl.ds(0, 16)][0] >> 7
        bhi = sidx_v[pl.ds(_BPW - 16, 16)][15] >> 7

        lane16 = jnp.arange(16, dtype=jnp.int32)

        def start(b, i):
            off = pl.multiple_of(b * _LANES, _LANES)
            pltpu.async_copy(tableT_hbm.at[:, pl.ds(off, _LANES)],
                             ring_v.at[i], sems[i])

        def wait(i):
            pltpu.make_async_copy(
                tableT_hbm.at[:, pl.ds(0, _LANES)], ring_v.at[i],
                sems[i]).wait()

        def proc(i, b, q0):
            blk = ring_v.at[i]
            lim = (b + 1) * _LANES

            def cond(q):
                v = sidx_v[pl.ds(q, 16)][0]
                return jnp.logical_and(q < _BPW, v < lim)

            def body(q):
                v = sidx_v[pl.ds(q, 16)][0]
                o = opos_v[pl.ds(q, 16)][0]
                lvec = jnp.full((16,), 0, jnp.int32) + (v & (_LANES - 1))
                for g in range(4):
                    col = plsc.load_gather(
                        blk, [lane16 + g * 16, lvec])
                    rows_v[q, pl.ds(g * 16, 16)] = col
                pltpu.async_copy(rows_v.at[q], out_hbm.at[o], semw)
                return q + 1

            return lax.while_loop(cond, body, q0)

        for i in range(_RING):
            @pl.when(blo + i <= bhi)
            def _(i=i):
                start(blo + i, i)

        def rotate(k, q):
            b0 = blo + k * _RING
            for i in range(_RING):
                b = b0 + i

                @pl.when(b <= bhi)
                def _(i=i):
                    wait(i)

                q = proc(i, b, q)

                @pl.when(b + _RING <= bhi)
                def _(i=i, b=b):
                    start(b + _RING, i)

            return q

        nrot = (bhi - blo + _RING) // _RING
        lax.fori_loop(0, nrot, rotate, 0)

        pltpu.make_async_copy(
            out_hbm.at[pl.ds(0, _BPW)], rows_v, semw).wait()

    return gather_kernel


def kernel(memory, index, t, current_event_id):
    idx = index.astype(jnp.int32)
    pos = jnp.arange(_B, dtype=jnp.int32)
    sidx, opos = lax.sort((idx, pos), num_keys=1)
    return _build()(memory.T, sidx, opos)
